# baseline (speedup 1.0000x reference)
import jax
import jax.numpy as jnp
from jax import lax
from jax.experimental import pallas as pl
from jax.experimental.pallas import tpu as pltpu
from jax.experimental.pallas import tpu_sc as plsc

N_PS = 50000
N_GW = 50000
N_SW = 50000
E = 150000
D = 128
OUT = 128
DA = 128

NC, NS, LANES = 2, 16, 16
B = 128
EPT = 9472
E_PAD = EPT * NS
NB = EPT // B
N_CHUNK = 18
CHS = 2784
N_OUT_PAD = N_CHUNK * CHS
DUMP = CHS
ACC_ROWS = 2816
RPT = ACC_ROWS // NS
FL0 = CHS - (NS - 1) * RPT

ROW_BLK = 1000


def _sc_body(x_ps, x_gw, x_sw,
             s1, d1, s2, d2, s3, d3, s4, d4,
             o_s1, o_s2, o_s3, o_s4,
             idx_src_v, idx_dst_v, comp_src, comp_dst, rows_v, zero_v,
             acc_sh):
    core = lax.axis_index("c")
    sid = lax.axis_index("s")
    base = sid * RPT

    z16f = jnp.zeros((LANES,), jnp.float32)

    @pl.loop(0, B)
    def _(r):
        for c in range(DA // LANES):
            zero_v[r, pl.ds(c * LANES, LANES)] = z16f

    def run_edge_type(src_hbm, dst_hbm, x_hbm, out_s):
        pltpu.sync_copy(src_hbm.at[pl.ds(sid * EPT, EPT)], idx_src_v)
        pltpu.sync_copy(dst_hbm.at[pl.ds(sid * EPT, EPT)], idx_dst_v)

        @pl.loop(0, N_CHUNK // NC)
        def _(j):
            chunk = (N_CHUNK // NC) * core + j
            lo = chunk * CHS

            pltpu.sync_copy(zero_v, acc_sh.at[pl.ds(base, B)])
            pltpu.sync_copy(zero_v.at[pl.ds(0, RPT - B)],
                            acc_sh.at[pl.ds(base + B, RPT - B)])
            plsc.subcore_barrier()

            def comp_body(r, carry):
                for c in range(B // LANES):
                    off = r * B + c * LANES
                    dstv = idx_dst_v[pl.ds(off, LANES)]
                    srcv = idx_src_v[pl.ds(off, LANES)]
                    ldst = dstv - lo
                    mask = (ldst >= 0) & (ldst < CHS)
                    comp_src[r, pl.ds(c * LANES, LANES)] = jnp.where(
                        mask, srcv, 0)
                    comp_dst[r, pl.ds(c * LANES, LANES)] = jnp.where(
                        mask, ldst, DUMP)
                return carry

            lax.fori_loop(0, NB, comp_body, jnp.int32(0))

            def gs_body(g, carry):
                pltpu.sync_copy(x_hbm.at[comp_src.at[g]], rows_v)
                pltpu.sync_copy(rows_v, acc_sh.at[comp_dst.at[g]], add=True)
                return carry

            lax.fori_loop(0, NB, gs_body, jnp.int32(0))
            plsc.subcore_barrier()

            out_row = lo + base
            pltpu.sync_copy(acc_sh.at[pl.ds(base, FL0)],
                            out_s.at[pl.ds(out_row, FL0)])

            @pl.when(sid < NS - 1)
            def _():
                pltpu.sync_copy(acc_sh.at[pl.ds(base + FL0, RPT - FL0)],
                                out_s.at[pl.ds(out_row + FL0, RPT - FL0)])

    run_edge_type(s1, d1, x_ps, o_s1)
    run_edge_type(s2, d2, x_gw, o_s2)
    run_edge_type(s3, d3, x_ps, o_s3)
    run_edge_type(s4, d4, x_sw, o_s4)


def _sc_segment_sums(x_ps, x_gw, x_sw, edges):
    mesh = plsc.VectorSubcoreMesh(core_axis_name="c", subcore_axis_name="s",
                                  num_cores=NC, num_subcores=NS)
    f32 = jnp.float32
    outs = [jax.ShapeDtypeStruct((N_OUT_PAD, DA), f32) for _ in range(4)]
    kern = pl.kernel(
        _sc_body,
        out_type=outs,
        mesh=mesh,
        compiler_params=pltpu.CompilerParams(needs_layout_passes=False),
        scratch_types=[
            pltpu.VMEM((EPT,), jnp.int32),
            pltpu.VMEM((EPT,), jnp.int32),
            pltpu.VMEM((NB, B), jnp.int32),
            pltpu.VMEM((NB, B), jnp.int32),
            pltpu.VMEM((B, DA), f32),
            pltpu.VMEM((B, DA), f32),
            pltpu.VMEM_SHARED((ACC_ROWS, DA), f32),
        ],
    )
    flat_edges = []
    for s, d in edges:
        flat_edges += [s, d]
    return kern(x_ps, x_gw, x_sw, *flat_edges)


def _post_body(x_ps, x_gw, x_sw,
               s1, c1, s2, c2, s3, c3, s4, c4,
               wl1, wl2, wl3, wl4, wr1, wr24, wr3,
               bgw, bps, bsw, wlin, misc,
               out_ps, out_gwlin, out_sw):
    def mean(s_ref, c_ref):
        return s_ref[...] / jnp.maximum(c_ref[...], 1.0)

    m1 = mean(s1, c1)
    h_gw = (jnp.dot(m1, wl1[...], preferred_element_type=jnp.float32)
            + jnp.dot(x_gw[...], wr1[...], preferred_element_type=jnp.float32)
            + bgw[...])
    r_gw = jnp.maximum(h_gw, 0.0)
    blin = misc[0, 0]
    a = misc[0, 1]
    g = jnp.sum(r_gw * wlin[...], axis=1, keepdims=True) + blin
    out_gwlin[...] = jnp.where(g >= 0, g, a * g)

    m2 = mean(s2, c2)
    m4 = mean(s4, c4)
    h_ps = (jnp.dot(m2, wl2[...], preferred_element_type=jnp.float32)
            + jnp.dot(m4, wl4[...], preferred_element_type=jnp.float32)
            + jnp.dot(x_ps[...], wr24[...], preferred_element_type=jnp.float32)
            + bps[...])
    out_ps[...] = jnp.maximum(h_ps, 0.0)

    m3 = mean(s3, c3)
    h_sw = (jnp.dot(m3, wl3[...], preferred_element_type=jnp.float32)
            + jnp.dot(x_sw[...], wr3[...], preferred_element_type=jnp.float32)
            + bsw[...])
    out_sw[...] = jnp.maximum(h_sw, 0.0)


def _post(x_ps, x_gw, x_sw, s1, c1, s2, c2, s3, c3, s4, c4,
          wl1, wl2, wl3, wl4, wr1, wr24, wr3, bgw, bps, bsw, wlin, misc):
    n = N_PS
    grid = (n // ROW_BLK,)
    row = pl.BlockSpec((ROW_BLK, D), lambda i: (i, 0))
    srow = pl.BlockSpec((ROW_BLK, DA), lambda i: (i, 0))
    crow = pl.BlockSpec((ROW_BLK, 1), lambda i: (i, 0))
    w = pl.BlockSpec((D, D), lambda i: (0, 0))
    b = pl.BlockSpec((1, D), lambda i: (0, 0))
    return pl.pallas_call(
        _post_body,
        grid=grid,
        in_specs=[row, row, row,
                  srow, crow, srow, crow, srow, crow, srow, crow,
                  w, w, w, w, w, w, w,
                  b, b, b, b, b],
        out_specs=[row,
                   pl.BlockSpec((ROW_BLK, 1), lambda i: (i, 0)),
                   row],
        out_shape=[jax.ShapeDtypeStruct((n, OUT), jnp.float32),
                   jax.ShapeDtypeStruct((n, 1), jnp.float32),
                   jax.ShapeDtypeStruct((n, OUT), jnp.float32)],
    )(x_ps, x_gw, x_sw, s1, c1, s2, c2, s3, c3, s4, c4,
      wl1, wl2, wl3, wl4, wr1, wr24, wr3, bgw, bps, bsw, wlin, misc)


def kernel(x_pfas_sites, x_gw_wells, x_sw_stations, ei_ps_gw, ei_gw_ps,
           ei_ps_sw, ei_sw_ps, Wl1, bl1, Wr1, Wl2, bl2, Wr2, Wl3, bl3, Wr3,
           Wl4, bl4, Wr4, Wlin, blin, prelu_a):
    def aug(x):
        n = x.shape[0]
        return jnp.concatenate(
            [x, jnp.ones((n, 1), jnp.float32),
             jnp.zeros((n, DA - D - 1), jnp.float32)], axis=1)

    def prep(ei):
        src = jnp.concatenate(
            [ei[0].astype(jnp.int32), jnp.zeros((E_PAD - E,), jnp.int32)])
        dst = jnp.concatenate(
            [ei[1].astype(jnp.int32),
             jnp.full((E_PAD - E,), jnp.int32(1 << 30))])
        return src, dst

    edges = [prep(ei) for ei in (ei_ps_gw, ei_gw_ps, ei_ps_sw, ei_sw_ps)]
    s1, s2, s3, s4 = _sc_segment_sums(
        x_pfas_sites, x_gw_wells, x_sw_stations, edges)

    def cnt(ei, n):
        c = jax.ops.segment_sum(
            jnp.ones((E,), jnp.float32), ei[1], num_segments=n)
        return c.reshape(n, 1)

    c1 = cnt(ei_ps_gw, N_GW)
    c2 = cnt(ei_gw_ps, N_PS)
    c3 = cnt(ei_ps_sw, N_SW)
    c4 = cnt(ei_sw_ps, N_PS)

    misc = jnp.stack([blin[0], prelu_a]).reshape(1, 2)
    misc = jnp.pad(misc, ((0, 0), (0, D - 2)))
    out_ps, gw, out_sw = _post(
        x_pfas_sites, x_gw_wells, x_sw_stations,
        s1, c1, s2, c2, s3, c3, s4, c4,
        Wl1.T, Wl2.T, Wl3.T, Wl4.T, Wr1.T, (Wr2 + Wr4).T, Wr3.T,
        bl1.reshape(1, D), (bl2 + bl4).reshape(1, D), bl3.reshape(1, D),
        Wlin.reshape(1, OUT), misc)
    return (out_ps, gw, out_sw)


# speedup vs baseline: 51.0996x; 51.0996x over previous
import jax
import jax.numpy as jnp
from jax import lax
from jax.experimental import pallas as pl
from jax.experimental.pallas import tpu as pltpu
from jax.experimental.pallas import tpu_sc as plsc

N_PS = 50000
N_GW = 50000
N_SW = 50000
E = 150000
D = 128
OUT = 128
DA = 128

NC, NS, LANES = 2, 16, 16
B = 128
EPT = 9472
E_PAD = EPT * NS
NB = EPT // B
VREGS = EPT // LANES
CROWS = VREGS + B // LANES
ZROWS = 56
N_CHUNK = 31
CHS = 1664
N_OUT_PAD = N_CHUNK * CHS
CPC0 = 16
DUMP = CHS
ACC_ROWS = 1792
RPT = ACC_ROWS // NS
FL0 = 96
CNTR = CHS // LANES

ROW_BLK = 2048

def _rot16(x, idx):
    dn = lax.GatherDimensionNumbers(offset_dims=(), collapsed_slice_dims=(0,),
                                    start_index_map=(0,))
    return lax.gather(x, idx[:, None], dn, (1,),
                      mode=lax.GatherScatterMode.PROMISE_IN_BOUNDS)


def _sc_body(x_ps, x_gw, x_sw,
             s1, d1, s2, d2, s3, d3, s4, d4,
             o_s1, o_c1, o_s2, o_c2, o_s3, o_c3, o_s4, o_c4,
             idx_src_v, idx_dst_v, comp_p, rowbuf, idx_bs, idx_bd, rows_v,
             zero_v, cnt_loc, acc_sh):
    core = lax.axis_index("c")
    sid = lax.axis_index("s")
    base = sid * RPT
    lane = lax.iota(jnp.int32, LANES)

    z16f = jnp.zeros((LANES,), jnp.float32)

    @pl.loop(0, ZROWS)
    def _(r):
        for c in range(DA // LANES):
            zero_v[r, pl.ds(c * LANES, LANES)] = z16f

    dump_packed = (DUMP + lane) * 65536

    def run_edge_type(src_hbm, dst_hbm, x_hbm, out_s, out_c):
        pltpu.sync_copy(src_hbm.at[pl.ds(sid * EPT, EPT)], idx_src_v)
        pltpu.sync_copy(dst_hbm.at[pl.ds(sid * EPT, EPT)], idx_dst_v)
        ncpc = jnp.where(core == 0, CPC0, N_CHUNK - CPC0)

        @pl.loop(0, ncpc)
        def _(j):
            chunk = core * CPC0 + j
            lo = chunk * CHS

            pltpu.sync_copy(zero_v, acc_sh.at[pl.ds(base, ZROWS)])
            pltpu.sync_copy(zero_v, acc_sh.at[pl.ds(base + ZROWS, ZROWS)])

            @pl.loop(0, CNTR + 2)
            def _(r):
                cnt_loc[r, pl.ds(0, LANES)] = z16f
            plsc.subcore_barrier()

            def flush_rowbuf(w):
                for k in range(B // LANES):
                    comp_p[w, pl.ds(k * LANES, LANES)] = (
                        rowbuf[k, pl.ds(0, LANES)])

            def comp_body(v, carry):
                w, q, m, pend = carry
                dstv = idx_dst_v[pl.ds(v * LANES, LANES)]
                srcv = idx_src_v[pl.ds(v * LANES, LANES)]
                ldst = dstv - lo
                mask = (ldst >= 0) & (ldst < CHS)
                k = jnp.sum(mask.astype(jnp.int32))
                packed = jnp.where(mask, srcv + ldst * 65536, 0)
                key = jnp.where(mask, 0, 1)
                _, sv = plsc.sort_key_val(key, packed)
                rot = _rot16(sv, (lane + (LANES - m)) & (LANES - 1))
                merged = jnp.where(lane < m, pend, rot)
                total = m + k
                full = total >= LANES

                @pl.when(full)
                def _():
                    rowbuf[q, pl.ds(0, LANES)] = merged

                fi = full.astype(jnp.int32)
                q2 = q + fi
                rb_full = q2 >= B // LANES

                @pl.when(rb_full)
                def _():
                    flush_rowbuf(w)

                ri = rb_full.astype(jnp.int32)
                w2 = w + ri
                q3 = q2 - (B // LANES) * ri
                m2 = total - LANES * fi
                pend2 = jnp.where(full, rot, merged)
                return w2, q3, m2, pend2

            w, q, m, pend = lax.fori_loop(
                0, VREGS, comp_body,
                (jnp.int32(0), jnp.int32(0), jnp.int32(0),
                 jnp.zeros((LANES,), jnp.int32)))

            tail = jnp.where(lane < m, pend, dump_packed)
            rowbuf[q, pl.ds(0, LANES)] = tail

            @pl.loop(q + 1, B // LANES)
            def _(k):
                rowbuf[k, pl.ds(0, LANES)] = dump_packed

            flush_rowbuf(w)
            nb = w + 1

            def gs_body(g, carry):
                for c in range(B // LANES):
                    p = comp_p[g, pl.ds(c * LANES, LANES)]
                    ldv = lax.shift_right_logical(p, 16)
                    idx_bs[0, pl.ds(c * LANES, LANES)] = p & 65535
                    idx_bd[0, pl.ds(c * LANES, LANES)] = ldv
                    for t in range(LANES):
                        ldt = ldv[t]
                        r = lax.shift_right_logical(ldt, 4)
                        oh = (lane == (ldt & 15)).astype(jnp.float32)
                        cnt_loc[r, pl.ds(0, LANES)] = (
                            cnt_loc[r, pl.ds(0, LANES)] + oh)
                pltpu.sync_copy(x_hbm.at[idx_bs.at[0]], rows_v)
                pltpu.sync_copy(rows_v, acc_sh.at[idx_bd.at[0]], add=True)
                return carry

            lax.fori_loop(0, nb, gs_body, jnp.int32(0))
            plsc.subcore_barrier()

            out_row = lo + base

            @pl.when(sid < NS - 1)
            def _():
                pltpu.sync_copy(acc_sh.at[pl.ds(base, FL0)],
                                out_s.at[pl.ds(out_row, FL0)])

            @pl.when(sid < NS - 2)
            def _():
                pltpu.sync_copy(acc_sh.at[pl.ds(base + FL0, RPT - FL0)],
                                out_s.at[pl.ds(out_row + FL0, RPT - FL0)])

            pltpu.sync_copy(cnt_loc.at[pl.ds(0, CNTR)], out_c.at[sid, chunk])

    run_edge_type(s1, d1, x_ps, o_s1, o_c1)
    run_edge_type(s2, d2, x_gw, o_s2, o_c2)
    run_edge_type(s3, d3, x_ps, o_s3, o_c3)
    run_edge_type(s4, d4, x_sw, o_s4, o_c4)


def _sc_segment_sums(x_ps, x_gw, x_sw, edges):
    mesh = plsc.VectorSubcoreMesh(core_axis_name="c", subcore_axis_name="s",
                                  num_cores=NC, num_subcores=NS)
    f32 = jnp.float32
    outs = []
    for _ in range(4):
        outs.append(jax.ShapeDtypeStruct((N_OUT_PAD, DA), f32))
        outs.append(jax.ShapeDtypeStruct((NS, N_CHUNK, CNTR, LANES), f32))
    kern = pl.kernel(
        _sc_body,
        out_type=outs,
        mesh=mesh,
        compiler_params=pltpu.CompilerParams(needs_layout_passes=False),
        scratch_types=[
            pltpu.VMEM((EPT,), jnp.int32),
            pltpu.VMEM((EPT,), jnp.int32),
            pltpu.VMEM((CROWS // (B // LANES) + 1, B), jnp.int32),
            pltpu.VMEM((B // LANES, LANES), jnp.int32),
            pltpu.VMEM((1, B), jnp.int32),
            pltpu.VMEM((1, B), jnp.int32),
            pltpu.VMEM((B, DA), f32),
            pltpu.VMEM((ZROWS, DA), f32),
            pltpu.VMEM((CNTR + 2, LANES), f32),
            pltpu.VMEM_SHARED((ACC_ROWS, DA), f32),
        ],
    )
    flat_edges = []
    for s, d in edges:
        flat_edges += [s, d]
    return kern(x_ps, x_gw, x_sw, *flat_edges)


def _post_body(x_ps, x_gw, x_sw,
               s1, c1, s2, c2, s3, c3, s4, c4,
               wl1, wl2, wl3, wl4, wr1, wr24, wr3,
               bgw, bps, bsw, wlin, misc,
               out_ps, out_gwlin, out_sw):
    def mean(s_ref, c_ref):
        cnt = jnp.sum(c_ref[...], axis=0)[:, None]
        return s_ref[...] / jnp.maximum(cnt, 1.0)

    m1 = mean(s1, c1)
    h_gw = (jnp.dot(m1, wl1[...], preferred_element_type=jnp.float32)
            + jnp.dot(x_gw[...], wr1[...], preferred_element_type=jnp.float32)
            + bgw[...])
    r_gw = jnp.maximum(h_gw, 0.0)
    blin = misc[0, 0]
    a = misc[0, 1]
    g = jnp.sum(r_gw * wlin[...], axis=1, keepdims=True) + blin
    out_gwlin[...] = jnp.where(g >= 0, g, a * g)

    m2 = mean(s2, c2)
    m4 = mean(s4, c4)
    h_ps = (jnp.dot(m2, wl2[...], preferred_element_type=jnp.float32)
            + jnp.dot(m4, wl4[...], preferred_element_type=jnp.float32)
            + jnp.dot(x_ps[...], wr24[...], preferred_element_type=jnp.float32)
            + bps[...])
    out_ps[...] = jnp.maximum(h_ps, 0.0)

    m3 = mean(s3, c3)
    h_sw = (jnp.dot(m3, wl3[...], preferred_element_type=jnp.float32)
            + jnp.dot(x_sw[...], wr3[...], preferred_element_type=jnp.float32)
            + bsw[...])
    out_sw[...] = jnp.maximum(h_sw, 0.0)


def _post(x_ps, x_gw, x_sw, s1, c1, s2, c2, s3, c3, s4, c4,
          wl1, wl2, wl3, wl4, wr1, wr24, wr3, bgw, bps, bsw, wlin, misc):
    n = N_PS
    grid = (pl.cdiv(n, ROW_BLK),)
    row = pl.BlockSpec((ROW_BLK, D), lambda i: (i, 0))
    srow = pl.BlockSpec((ROW_BLK, DA), lambda i: (i, 0))
    crow = pl.BlockSpec((NS, ROW_BLK), lambda i: (0, i))
    w = pl.BlockSpec((D, D), lambda i: (0, 0))
    b = pl.BlockSpec((1, D), lambda i: (0, 0))
    return pl.pallas_call(
        _post_body,
        grid=grid,
        in_specs=[row, row, row,
                  srow, crow, srow, crow, srow, crow, srow, crow,
                  w, w, w, w, w, w, w,
                  b, b, b, b, b],
        out_specs=[row,
                   pl.BlockSpec((ROW_BLK, 1), lambda i: (i, 0)),
                   row],
        out_shape=[jax.ShapeDtypeStruct((n, OUT), jnp.float32),
                   jax.ShapeDtypeStruct((n, 1), jnp.float32),
                   jax.ShapeDtypeStruct((n, OUT), jnp.float32)],
    )(x_ps, x_gw, x_sw, s1, c1, s2, c2, s3, c3, s4, c4,
      wl1, wl2, wl3, wl4, wr1, wr24, wr3, bgw, bps, bsw, wlin, misc)


def kernel(x_pfas_sites, x_gw_wells, x_sw_stations, ei_ps_gw, ei_gw_ps,
           ei_ps_sw, ei_sw_ps, Wl1, bl1, Wr1, Wl2, bl2, Wr2, Wl3, bl3, Wr3,
           Wl4, bl4, Wr4, Wlin, blin, prelu_a):
    def aug(x):
        n = x.shape[0]
        return jnp.concatenate(
            [x, jnp.ones((n, 1), jnp.float32),
             jnp.zeros((n, DA - D - 1), jnp.float32)], axis=1)

    def prep(ei):
        src = jnp.concatenate(
            [ei[0].astype(jnp.int32), jnp.zeros((E_PAD - E,), jnp.int32)])
        dst = jnp.concatenate(
            [ei[1].astype(jnp.int32),
             jnp.full((E_PAD - E,), jnp.int32(1 << 30))])
        return src, dst

    edges = [prep(ei) for ei in (ei_ps_gw, ei_gw_ps, ei_ps_sw, ei_sw_ps)]
    s1, c1, s2, c2, s3, c3, s4, c4 = _sc_segment_sums(
        x_pfas_sites, x_gw_wells, x_sw_stations, edges)
    c1, c2, c3, c4 = (c.reshape(NS, N_OUT_PAD) for c in (c1, c2, c3, c4))

    misc = jnp.stack([blin[0], prelu_a]).reshape(1, 2)
    misc = jnp.pad(misc, ((0, 0), (0, D - 2)))
    out_ps, gw, out_sw = _post(
        x_pfas_sites, x_gw_wells, x_sw_stations,
        s1, c1, s2, c2, s3, c3, s4, c4,
        Wl1.T, Wl2.T, Wl3.T, Wl4.T, Wr1.T, (Wr2 + Wr4).T, Wr3.T,
        bl1.reshape(1, D), (bl2 + bl4).reshape(1, D), bl3.reshape(1, D),
        Wlin.reshape(1, OUT), misc)
    return (out_ps, gw, out_sw)
